# Initial kernel scaffold; baseline (speedup 1.0000x reference)
import jax
import jax.numpy as jnp
from jax.experimental import pallas as pl


def kernel(x, expert_mus, expert_log_sigmas, W1, b1, W2, b2):
    raise NotImplementedError("write your pallas kernel here")


# speedup vs baseline: 1.0917x; 1.0917x over previous
import functools
import math

import jax
import jax.numpy as jnp
from jax.experimental import pallas as pl
from jax.experimental.pallas import tpu as pltpu


def _router_body(x_ref, mus_ref, ls_ref, sig_ref, lp_ref, w_ref, idx_ref):
    x = x_ref[...]
    n_e = mus_ref.shape[0]
    c = 0.5 * math.log(2.0 * math.pi)
    cols = []
    for e in range(n_e):
        t = (x - mus_ref[e][None, :]) / sig_ref[e][None, :]
        term = -0.5 * (t * t) - ls_ref[e][None, :] - c
        cols.append(jnp.sum(term, axis=1, keepdims=True))
    lp = jnp.concatenate(cols, axis=1)
    lp_ref[...] = lp

    rows, e = lp.shape
    iota = jax.lax.broadcasted_iota(jnp.int32, (rows, e), 1)
    v1 = jnp.max(lp, axis=1, keepdims=True)
    i1 = jnp.min(jnp.where(lp == v1, iota, e), axis=1, keepdims=True)
    masked = jnp.where(iota == i1, -jnp.inf, lp)
    v2 = jnp.max(masked, axis=1, keepdims=True)
    i2 = jnp.min(jnp.where(masked == v2, iota, e), axis=1, keepdims=True)
    ew = jnp.exp(v2 - v1)
    w1 = 1.0 / (1.0 + ew)
    w2 = ew * w1
    w_ref[...] = jnp.concatenate([w1, w2], axis=1)
    idx_ref[...] = jnp.concatenate([i1, i2], axis=1)


def _erf(z):
    a1, a2, a3, a4, a5 = (0.254829592, -0.284496736, 1.421413741,
                          -1.453152027, 1.061405429)
    p = 0.3275911
    s = jnp.sign(z)
    az = jnp.abs(z)
    t = 1.0 / (1.0 + p * az)
    poly = ((((a5 * t + a4) * t + a3) * t + a2) * t + a1) * t
    return s * (1.0 - poly * jnp.exp(-az * az))


def _gelu(v):
    return 0.5 * v * (1.0 + _erf(v * (1.0 / math.sqrt(2.0))))


def _mlp_body(x_ref, w1_ref, w2_ref, wts_ref, idx_ref, y_ref, acc_ref,
              *, n_e, n_h):
    e = pl.program_id(1)
    h = pl.program_id(2)

    ht = _gelu(jnp.dot(x_ref[...], w1_ref[0], preferred_element_type=jnp.float32))
    part = jnp.dot(ht, w2_ref[0], preferred_element_type=jnp.float32)

    wts = wts_ref[...]
    idx = idx_ref[...]
    w_col = jnp.sum(jnp.where(idx == e, wts, 0.0), axis=1, keepdims=True)
    contrib = w_col * part

    @pl.when(jnp.logical_and(e == 0, h == 0))
    def _init():
        acc_ref[...] = contrib

    @pl.when(jnp.logical_not(jnp.logical_and(e == 0, h == 0)))
    def _acc():
        acc_ref[...] += contrib

    @pl.when(jnp.logical_and(e == n_e - 1, h == n_h - 1))
    def _out():
        y_ref[...] = acc_ref[...]


def kernel(x, expert_mus, expert_log_sigmas, W1, b1, W2, b2):
    Bn, S, D = x.shape
    E, _, H = W1.shape
    O = W2.shape[-1]
    N = Bn * S
    x_flat = x.reshape(N, D)

    sigma = jnp.exp(expert_log_sigmas)

    blk_s = 512 if N % 512 == 0 else N
    n_s = N // blk_s

    lp, wts, tidx = pl.pallas_call(
        _router_body,
        grid=(n_s,),
        in_specs=[
            pl.BlockSpec((blk_s, D), lambda i: (i, 0)),
            pl.BlockSpec((E, D), lambda i: (0, 0)),
            pl.BlockSpec((E, D), lambda i: (0, 0)),
            pl.BlockSpec((E, D), lambda i: (0, 0)),
        ],
        out_specs=[
            pl.BlockSpec((blk_s, E), lambda i: (i, 0)),
            pl.BlockSpec((blk_s, 2), lambda i: (i, 0)),
            pl.BlockSpec((blk_s, 2), lambda i: (i, 0)),
        ],
        out_shape=[
            jax.ShapeDtypeStruct((N, E), jnp.float32),
            jax.ShapeDtypeStruct((N, 2), jnp.float32),
            jax.ShapeDtypeStruct((N, 2), jnp.int32),
        ],
    )(x_flat, expert_mus, expert_log_sigmas, sigma)

    pad = 8
    x_aug = jnp.concatenate(
        [x_flat, jnp.ones((N, 1), jnp.float32), jnp.zeros((N, pad - 1), jnp.float32)],
        axis=1)
    w1_aug = jnp.concatenate(
        [W1, b1[:, None, :], jnp.zeros((E, pad - 1, H), jnp.float32)], axis=1)
    Da = D + pad

    blk_h = 512 if H % 512 == 0 else H
    n_h = H // blk_h

    y = pl.pallas_call(
        functools.partial(_mlp_body, n_e=E, n_h=n_h),
        grid=(n_s, E, n_h),
        in_specs=[
            pl.BlockSpec((blk_s, Da), lambda i, e, h: (i, 0)),
            pl.BlockSpec((1, Da, blk_h), lambda i, e, h: (e, 0, h)),
            pl.BlockSpec((1, blk_h, O), lambda i, e, h: (e, h, 0)),
            pl.BlockSpec((blk_s, 2), lambda i, e, h: (i, 0)),
            pl.BlockSpec((blk_s, 2), lambda i, e, h: (i, 0)),
        ],
        out_specs=pl.BlockSpec((blk_s, O), lambda i, e, h: (i, 0)),
        out_shape=jax.ShapeDtypeStruct((N, O), jnp.float32),
        scratch_shapes=[pltpu.VMEM((blk_s, O), jnp.float32)],
        compiler_params=pltpu.CompilerParams(
            dimension_semantics=("arbitrary", "arbitrary", "arbitrary")),
    )(x_aug, w1_aug, W2, wts, tidx)

    wfull = jnp.sum(
        jnp.where(tidx[:, :, None] == jnp.arange(E)[None, None, :],
                  wts[:, :, None], 0.0), axis=1)
    y = y + wfull @ b2

    return (y.reshape(Bn, S, O),
            lp.reshape(Bn, S, E),
            wts.reshape(Bn, S, 2),
            tidx.reshape(Bn, S, 2))


# speedup vs baseline: 1.3885x; 1.2719x over previous
import functools
import math

import jax
import jax.numpy as jnp
from jax import lax
from jax.experimental import pallas as pl
from jax.experimental.pallas import tpu as pltpu
from jax.experimental.pallas import tpu_sc as plsc

L = 16
NT = 16
NBLK = 128
NBLK_LOG = NBLK.bit_length() - 1


def _router_body(x_ref, mus_ref, ls_ref, sig_ref, lp_ref, w_ref, idx_ref):
    x = x_ref[...]
    n_e = mus_ref.shape[0]
    c = 0.5 * math.log(2.0 * math.pi)
    cols = []
    for e in range(n_e):
        t = (x - mus_ref[e][None, :]) / sig_ref[e][None, :]
        term = -0.5 * (t * t) - ls_ref[e][None, :] - c
        cols.append(jnp.sum(term, axis=1, keepdims=True))
    lp = jnp.concatenate(cols, axis=1)
    lp_ref[...] = lp

    rows, e = lp.shape
    iota = lax.broadcasted_iota(jnp.int32, (rows, e), 1)
    v1 = jnp.max(lp, axis=1, keepdims=True)
    i1 = jnp.min(jnp.where(lp == v1, iota, e), axis=1, keepdims=True)
    masked = jnp.where(iota == i1, -jnp.inf, lp)
    v2 = jnp.max(masked, axis=1, keepdims=True)
    i2 = jnp.min(jnp.where(masked == v2, iota, e), axis=1, keepdims=True)
    ew = jnp.exp(v2 - v1)
    w1 = 1.0 / (1.0 + ew)
    w_ref[...] = jnp.concatenate([w1, ew * w1], axis=1)
    idx_ref[...] = jnp.concatenate([i1, i2], axis=1)


def _erf(z):
    a1, a2, a3, a4, a5 = (0.254829592, -0.284496736, 1.421413741,
                          -1.453152027, 1.061405429)
    p = 0.3275911
    s = jnp.sign(z)
    az = jnp.abs(z)
    t = 1.0 / (1.0 + p * az)
    poly = ((((a5 * t + a4) * t + a3) * t + a2) * t + a1) * t
    return s * (1.0 - poly * jnp.exp(-az * az))


def _gelu(v):
    return 0.5 * v * (1.0 + _erf(v * (1.0 / math.sqrt(2.0))))


def _lane_scalar(vec, e):
    io = lax.broadcasted_iota(jnp.int32, (L,), 0)
    return lax.reduce_max(jnp.where(io == e, vec, 0), axes=(0,))


def _count_body(ppt, n_e,
                tidx_hbm, counts_hbm, ranks_hbm,
                e2_vmem, rank_vmem, cnt_vmem, sem1):
    wid = lax.axis_index("s")
    io = lax.broadcasted_iota(jnp.int32, (L,), 0)
    pltpu.sync_copy(tidx_hbm.at[pl.ds(wid * (ppt // 2), ppt // 2)], e2_vmem)
    cnt = jnp.zeros((L,), jnp.int32)
    nv = ppt // L
    for v in range(nv):
        pj = v * L + io
        ev = plsc.load_gather(e2_vmem, [pj >> 1, pj & 1])
        pre = jnp.zeros((L,), jnp.int32)
        new_cnt = cnt
        for e in range(n_e):
            mi = (ev == e).astype(jnp.int32)
            cs = plsc.cumsum(mi)
            tot = lax.reduce_max(cs, axes=(0,))
            prior = _lane_scalar(cnt, e)
            pre = pre + mi * (cs - 1 + prior)
            new_cnt = new_cnt + tot * (io == e).astype(jnp.int32)
        cnt = new_cnt
        rank_vmem[pl.ds(v * L, L)] = pre
    cnt_vmem[...] = cnt
    pltpu.sync_copy(cnt_vmem, counts_hbm.at[wid])
    pltpu.sync_copy(rank_vmem, ranks_hbm.at[wid])


def _dispatch_body(ppt, nch, n_e, gmax, nbe,
                   tidx_hbm, x_hbm, counts_hbm, ranks_hbm,
                   xs_hbm, pos_hbm, bex_hbm,
                   e2_vmem, rank_vmem, o_vmem, pf_vmem,
                   allc_vmem, pos2_vmem, tok2_vmem, xbuf, bex_vmem,
                   sem1, sem2):
    wid = lax.axis_index("s")
    base = wid * ppt
    io = lax.broadcasted_iota(jnp.int32, (L,), 0)

    pltpu.sync_copy(tidx_hbm.at[pl.ds(wid * (ppt // 2), ppt // 2)], e2_vmem)
    pltpu.sync_copy(ranks_hbm.at[wid], rank_vmem)
    pltpu.sync_copy(counts_hbm, allc_vmem)

    prefix = jnp.zeros((L,), jnp.int32)
    total = jnp.zeros((L,), jnp.int32)
    for r in range(NT):
        row = allc_vmem[r]
        fl = (jnp.int32(r) < wid).astype(jnp.int32)
        prefix = prefix + row * fl
        total = total + row
    g_vec = ((total + (NBLK - 1)) >> NBLK_LOG) << NBLK_LOG
    o_vec = plsc.cumsum(g_vec) - g_vec
    end_vec = o_vec + g_vec
    o_vmem[...] = o_vec
    pf_vmem[...] = prefix

    nv = ppt // L
    for v in range(nv):
        pj = v * L + io
        ev = plsc.load_gather(e2_vmem, [pj >> 1, pj & 1])
        og = plsc.load_gather(o_vmem, [ev])
        pg = plsc.load_gather(pf_vmem, [ev])
        pos_v = og + pg + rank_vmem[pl.ds(v * L, L)]
        c2, half = divmod(v, 2)
        pos2_vmem[c2, pl.ds(half * L, L)] = pos_v
        tok2_vmem[c2, pl.ds(half * L, L)] = (base + v * L + io) >> 1

    pltpu.sync_copy(pos2_vmem, pos_hbm.at[wid])

    for c in range(nch):
        pltpu.async_copy(x_hbm.at[tok2_vmem.at[c]], xbuf, sem1).wait()
        pltpu.async_copy(xbuf, xs_hbm.at[pos2_vmem.at[c]], sem2).wait()

    @pl.when(wid == 0)
    def _():
        for k in range(nbe // L):
            bstart = (k * L + io) << NBLK_LOG
            acc = jnp.zeros((L,), jnp.int32)
            for e in range(n_e):
                end_e = _lane_scalar(end_vec, e)
                acc = acc + (bstart >= end_e).astype(jnp.int32)
            bex_vmem[pl.ds(k * L, L)] = jnp.minimum(acc, n_e - 1)
        pltpu.sync_copy(bex_vmem, bex_hbm)


def _mlp_body(bex_ref, xs_ref, w1_ref, b1_ref, w2_ref, b2_ref, out_ref,
              acc_ref, *, n_h):
    h = pl.program_id(1)
    ht = _gelu(jnp.dot(xs_ref[...], w1_ref[0],
                       preferred_element_type=jnp.float32) + b1_ref[0, 0])
    part = jnp.dot(ht, w2_ref[0], preferred_element_type=jnp.float32)

    @pl.when(h == 0)
    def _init():
        acc_ref[...] = part

    @pl.when(h != 0)
    def _acc():
        acc_ref[...] += part

    @pl.when(h == n_h - 1)
    def _out():
        out_ref[...] = acc_ref[...] + b2_ref[0]


def _combine_body(tpt, nch, dmodel,
                  op_hbm, pos_hbm, w_hbm, y_hbm,
                  pos2_vmem, w_vmem, rows_v, ybuf, sem1):
    wid = lax.axis_index("s")
    ct = tpt // nch
    pltpu.sync_copy(pos_hbm.at[wid], pos2_vmem)
    pltpu.sync_copy(w_hbm.at[pl.ds(wid * tpt, tpt)], w_vmem)
    nd = dmodel // L
    io = lax.broadcasted_iota(jnp.int32, (L,), 0)
    for c in range(nch):
        pltpu.async_copy(op_hbm.at[pos2_vmem.at[c]], rows_v, sem1).wait()
        wrow0 = plsc.load_gather(w_vmem, [c * ct + io,
                                          jnp.zeros((L,), jnp.int32)])
        wrow1 = plsc.load_gather(w_vmem, [c * ct + io,
                                          jnp.ones((L,), jnp.int32)])
        for t in range(ct):
            w0 = wrow0[t]
            w1 = wrow1[t]

            def body(d, carry):
                off = d * L
                a = rows_v[2 * t, pl.ds(off, L)]
                b = rows_v[2 * t + 1, pl.ds(off, L)]
                ybuf[t, pl.ds(off, L)] = w0 * a + w1 * b
                return carry

            lax.fori_loop(0, nd, body, 0)
        pltpu.sync_copy(ybuf, y_hbm.at[pl.ds(wid * tpt + c * ct, ct)])


def kernel(x, expert_mus, expert_log_sigmas, W1, b1, W2, b2):
    Bn, S, D = x.shape
    E, _, H = W1.shape
    O = W2.shape[-1]
    N = Bn * S
    K = 2
    P = N * K
    x_flat = x.reshape(N, D)

    gmax = -(-(P + E * (NBLK - 1)) // NBLK) * NBLK
    nb = gmax // NBLK
    nbe = -(-nb // L) * L
    ppt = P // NT
    nch = ppt // 32
    sigma = jnp.exp(expert_log_sigmas)

    blk_s = 512 if N % 512 == 0 else N
    n_s = N // blk_s
    lp, wts, tidx = pl.pallas_call(
        _router_body,
        grid=(n_s,),
        in_specs=[
            pl.BlockSpec((blk_s, D), lambda i: (i, 0)),
            pl.BlockSpec((E, D), lambda i: (0, 0)),
            pl.BlockSpec((E, D), lambda i: (0, 0)),
            pl.BlockSpec((E, D), lambda i: (0, 0)),
        ],
        out_specs=[
            pl.BlockSpec((blk_s, E), lambda i: (i, 0)),
            pl.BlockSpec((blk_s, 2), lambda i: (i, 0)),
            pl.BlockSpec((blk_s, 2), lambda i: (i, 0)),
        ],
        out_shape=[
            jax.ShapeDtypeStruct((N, E), jnp.float32),
            jax.ShapeDtypeStruct((N, 2), jnp.float32),
            jax.ShapeDtypeStruct((N, 2), jnp.int32),
        ],
    )(x_flat, expert_mus, expert_log_sigmas, sigma)

    mesh = plsc.VectorSubcoreMesh(core_axis_name="c", subcore_axis_name="s",
                                  num_cores=1, num_subcores=NT)
    counts, ranks = pl.kernel(
        functools.partial(_count_body, ppt, E),
        out_type=[
            jax.ShapeDtypeStruct((NT, L), jnp.int32),
            jax.ShapeDtypeStruct((NT, ppt), jnp.int32),
        ],
        mesh=mesh,
        scratch_types=[
            pltpu.VMEM((ppt // 2, 2), jnp.int32),
            pltpu.VMEM((ppt,), jnp.int32),
            pltpu.VMEM((L,), jnp.int32),
            pltpu.SemaphoreType.DMA,
        ],
        compiler_params=pltpu.CompilerParams(needs_layout_passes=False),
    )(tidx)

    xs, pos3, bex = pl.kernel(
        functools.partial(_dispatch_body, ppt, nch, E, gmax, nbe),
        out_type=[
            jax.ShapeDtypeStruct((gmax, D), jnp.float32),
            jax.ShapeDtypeStruct((NT, nch, 32), jnp.int32),
            jax.ShapeDtypeStruct((nbe,), jnp.int32),
        ],
        mesh=mesh,
        scratch_types=[
            pltpu.VMEM((ppt // 2, 2), jnp.int32),
            pltpu.VMEM((ppt,), jnp.int32),
            pltpu.VMEM((L,), jnp.int32),
            pltpu.VMEM((L,), jnp.int32),
            pltpu.VMEM((NT, L), jnp.int32),
            pltpu.VMEM((nch, 32), jnp.int32),
            pltpu.VMEM((nch, 32), jnp.int32),
            pltpu.VMEM((32, D), jnp.float32),
            pltpu.VMEM((nbe,), jnp.int32),
            pltpu.SemaphoreType.DMA,
            pltpu.SemaphoreType.DMA,
        ],
        compiler_params=pltpu.CompilerParams(needs_layout_passes=False),
    )(tidx, x_flat, counts, ranks)

    blk_h = 512 if H % 512 == 0 else H
    n_h = H // blk_h
    b1r = b1.reshape(E, n_h, 1, blk_h)
    b2r = b2.reshape(E, 1, O)

    out_pairs = pl.pallas_call(
        functools.partial(_mlp_body, n_h=n_h),
        grid_spec=pltpu.PrefetchScalarGridSpec(
            num_scalar_prefetch=1,
            grid=(nb, n_h),
            in_specs=[
                pl.BlockSpec((NBLK, D), lambda i, h, be: (i, 0)),
                pl.BlockSpec((1, D, blk_h), lambda i, h, be: (be[i], 0, h)),
                pl.BlockSpec((1, 1, 1, blk_h), lambda i, h, be: (be[i], h, 0, 0)),
                pl.BlockSpec((1, blk_h, O), lambda i, h, be: (be[i], h, 0)),
                pl.BlockSpec((1, 1, O), lambda i, h, be: (be[i], 0, 0)),
            ],
            out_specs=pl.BlockSpec((NBLK, O), lambda i, h, be: (i, 0)),
            scratch_shapes=[pltpu.VMEM((NBLK, O), jnp.float32)],
        ),
        out_shape=jax.ShapeDtypeStruct((gmax, O), jnp.float32),
        compiler_params=pltpu.CompilerParams(
            dimension_semantics=("arbitrary", "arbitrary")),
    )(bex, xs, W1, b1r, W2, b2r)

    tpt = N // NT
    y = pl.kernel(
        functools.partial(_combine_body, tpt, nch, O),
        out_type=jax.ShapeDtypeStruct((N, O), jnp.float32),
        mesh=mesh,
        scratch_types=[
            pltpu.VMEM((nch, 32), jnp.int32),
            pltpu.VMEM((tpt, 2), jnp.float32),
            pltpu.VMEM((32, O), jnp.float32),
            pltpu.VMEM((tpt // nch, O), jnp.float32),
            pltpu.SemaphoreType.DMA,
        ],
        compiler_params=pltpu.CompilerParams(needs_layout_passes=False),
    )(out_pairs, pos3, wts)

    return (y.reshape(Bn, S, O),
            lp.reshape(Bn, S, E),
            wts.reshape(Bn, S, 2),
            tidx.reshape(Bn, S, 2))


# speedup vs baseline: 1.4907x; 1.0736x over previous
import functools
import math

import jax
import jax.numpy as jnp
from jax import lax
from jax.experimental import pallas as pl
from jax.experimental.pallas import tpu as pltpu
from jax.experimental.pallas import tpu_sc as plsc

L = 16
NC = 2
NT = 32
NBLK = 128
NBLK_LOG = NBLK.bit_length() - 1


def _router_body(x_ref, mus_ref, ls_ref, sig_ref, lp_ref, w_ref, idx_ref):
    x = x_ref[...]
    n_e = mus_ref.shape[0]
    c = 0.5 * math.log(2.0 * math.pi)
    cols = []
    for e in range(n_e):
        t = (x - mus_ref[e][None, :]) / sig_ref[e][None, :]
        term = -0.5 * (t * t) - ls_ref[e][None, :] - c
        cols.append(jnp.sum(term, axis=1, keepdims=True))
    lp = jnp.concatenate(cols, axis=1)
    lp_ref[...] = lp

    rows, e = lp.shape
    iota = lax.broadcasted_iota(jnp.int32, (rows, e), 1)
    v1 = jnp.max(lp, axis=1, keepdims=True)
    i1 = jnp.min(jnp.where(lp == v1, iota, e), axis=1, keepdims=True)
    masked = jnp.where(iota == i1, -jnp.inf, lp)
    v2 = jnp.max(masked, axis=1, keepdims=True)
    i2 = jnp.min(jnp.where(masked == v2, iota, e), axis=1, keepdims=True)
    ew = jnp.exp(v2 - v1)
    w1 = 1.0 / (1.0 + ew)
    w_ref[...] = jnp.concatenate([w1, ew * w1], axis=1)
    idx_ref[...] = jnp.concatenate([i1, i2], axis=1)


def _erf(z):
    a1, a2, a3, a4, a5 = (0.254829592, -0.284496736, 1.421413741,
                          -1.453152027, 1.061405429)
    p = 0.3275911
    s = jnp.sign(z)
    az = jnp.abs(z)
    t = 1.0 / (1.0 + p * az)
    poly = ((((a5 * t + a4) * t + a3) * t + a2) * t + a1) * t
    return s * (1.0 - poly * jnp.exp(-az * az))


def _gelu(v):
    return 0.5 * v * (1.0 + _erf(v * (1.0 / math.sqrt(2.0))))


def _lane_scalar(vec, e):
    io = lax.broadcasted_iota(jnp.int32, (L,), 0)
    return lax.reduce_max(jnp.where(io == e, vec, 0), axes=(0,))


def _count_body(ppt, n_e,
                tidx_hbm, counts_hbm, ranks_hbm,
                e2_vmem, rank_vmem, cnt_vmem, sem1):
    wid = lax.axis_index("c") * (NT // NC) + lax.axis_index("s")
    io = lax.broadcasted_iota(jnp.int32, (L,), 0)
    pltpu.sync_copy(tidx_hbm.at[pl.ds(wid * (ppt // 2), ppt // 2)], e2_vmem)
    cnt = jnp.zeros((L,), jnp.int32)
    nv = ppt // L
    for v in range(nv):
        pj = v * L + io
        ev = plsc.load_gather(e2_vmem, [pj >> 1, pj & 1])
        pre = jnp.zeros((L,), jnp.int32)
        new_cnt = cnt
        for e in range(n_e):
            mi = (ev == e).astype(jnp.int32)
            cs = plsc.cumsum(mi)
            tot = lax.reduce_max(cs, axes=(0,))
            prior = _lane_scalar(cnt, e)
            pre = pre + mi * (cs - 1 + prior)
            new_cnt = new_cnt + tot * (io == e).astype(jnp.int32)
        cnt = new_cnt
        rank_vmem[pl.ds(v * L, L)] = pre
    cnt_vmem[...] = cnt
    pltpu.sync_copy(cnt_vmem, counts_hbm.at[wid])
    pltpu.sync_copy(rank_vmem, ranks_hbm.at[wid])


def _dispatch_body(ppt, nch, n_e, gmax, nbe,
                   tidx_hbm, x_hbm, counts_hbm, ranks_hbm,
                   xs_hbm, pos_hbm, bex_hbm,
                   e2_vmem, rank_vmem, o_vmem, pf_vmem,
                   allc_vmem, pos2_vmem, tok2_vmem, xbuf, bex_vmem,
                   sem1, sem2):
    wid = lax.axis_index("c") * (NT // NC) + lax.axis_index("s")
    base = wid * ppt
    io = lax.broadcasted_iota(jnp.int32, (L,), 0)

    pltpu.sync_copy(tidx_hbm.at[pl.ds(wid * (ppt // 2), ppt // 2)], e2_vmem)
    pltpu.sync_copy(ranks_hbm.at[wid], rank_vmem)
    pltpu.sync_copy(counts_hbm, allc_vmem)

    prefix = jnp.zeros((L,), jnp.int32)
    total = jnp.zeros((L,), jnp.int32)
    for r in range(NT):
        row = allc_vmem[r]
        fl = (jnp.int32(r) < wid).astype(jnp.int32)
        prefix = prefix + row * fl
        total = total + row
    g_vec = ((total + (NBLK - 1)) >> NBLK_LOG) << NBLK_LOG
    o_vec = plsc.cumsum(g_vec) - g_vec
    end_vec = o_vec + g_vec
    o_vmem[...] = o_vec
    pf_vmem[...] = prefix

    nv = ppt // L
    for v in range(nv):
        pj = v * L + io
        ev = plsc.load_gather(e2_vmem, [pj >> 1, pj & 1])
        og = plsc.load_gather(o_vmem, [ev])
        pg = plsc.load_gather(pf_vmem, [ev])
        pos_v = og + pg + rank_vmem[pl.ds(v * L, L)]
        c2, half = divmod(v, 2)
        pos2_vmem[c2, pl.ds(half * L, L)] = pos_v
        tok2_vmem[c2, pl.ds(half * L, L)] = (base + v * L + io) >> 1

    pltpu.sync_copy(pos2_vmem, pos_hbm.at[wid])

    for c in range(nch):
        pltpu.async_copy(x_hbm.at[tok2_vmem.at[c]], xbuf, sem1).wait()
        pltpu.async_copy(xbuf, xs_hbm.at[pos2_vmem.at[c]], sem2).wait()

    @pl.when(wid == 0)
    def _():
        for k in range(nbe // L):
            bstart = (k * L + io) << NBLK_LOG
            acc = jnp.zeros((L,), jnp.int32)
            for e in range(n_e):
                end_e = _lane_scalar(end_vec, e)
                acc = acc + (bstart >= end_e).astype(jnp.int32)
            bex_vmem[pl.ds(k * L, L)] = jnp.minimum(acc, n_e - 1)
        pltpu.sync_copy(bex_vmem, bex_hbm)


def _mlp_body(bex_ref, xs_ref, w1_ref, b1_ref, w2_ref, b2_ref, out_ref,
              acc_ref, *, n_h):
    h = pl.program_id(1)
    ht = _gelu(jnp.dot(xs_ref[...], w1_ref[0],
                       preferred_element_type=jnp.float32) + b1_ref[0, 0])
    part = jnp.dot(ht, w2_ref[0], preferred_element_type=jnp.float32)

    @pl.when(h == 0)
    def _init():
        acc_ref[...] = part

    @pl.when(h != 0)
    def _acc():
        acc_ref[...] += part

    @pl.when(h == n_h - 1)
    def _out():
        out_ref[...] = acc_ref[...] + b2_ref[0]


def _combine_body(tpt, nch, dmodel,
                  op_hbm, pos_hbm, w_hbm, y_hbm,
                  pos2_vmem, w_vmem, rows_v, ybuf, sem1):
    wid = lax.axis_index("c") * (NT // NC) + lax.axis_index("s")
    ct = tpt // nch
    pltpu.sync_copy(pos_hbm.at[wid], pos2_vmem)
    pltpu.sync_copy(w_hbm.at[pl.ds(wid * tpt, tpt)], w_vmem)
    nd = dmodel // L
    io = lax.broadcasted_iota(jnp.int32, (L,), 0)
    for c in range(nch):
        pltpu.async_copy(op_hbm.at[pos2_vmem.at[c]], rows_v, sem1).wait()
        wrow0 = plsc.load_gather(w_vmem, [c * ct + io,
                                          jnp.zeros((L,), jnp.int32)])
        wrow1 = plsc.load_gather(w_vmem, [c * ct + io,
                                          jnp.ones((L,), jnp.int32)])
        for t in range(ct):
            w0 = wrow0[t]
            w1 = wrow1[t]

            def body(d, carry):
                for u in range(4):
                    off = d * (4 * L) + u * L
                    a = rows_v[2 * t, pl.ds(off, L)]
                    b = rows_v[2 * t + 1, pl.ds(off, L)]
                    ybuf[t, pl.ds(off, L)] = w0 * a + w1 * b
                return carry

            lax.fori_loop(0, nd // 4, body, 0)
        pltpu.sync_copy(ybuf, y_hbm.at[pl.ds(wid * tpt + c * ct, ct)])


def kernel(x, expert_mus, expert_log_sigmas, W1, b1, W2, b2):
    Bn, S, D = x.shape
    E, _, H = W1.shape
    O = W2.shape[-1]
    N = Bn * S
    K = 2
    P = N * K
    x_flat = x.reshape(N, D)

    gmax = -(-(P + E * (NBLK - 1)) // NBLK) * NBLK
    nb = gmax // NBLK
    nbe = -(-nb // L) * L
    ppt = P // NT
    nch = ppt // 32
    sigma = jnp.exp(expert_log_sigmas)

    blk_s = 512 if N % 512 == 0 else N
    n_s = N // blk_s
    lp, wts, tidx = pl.pallas_call(
        _router_body,
        grid=(n_s,),
        in_specs=[
            pl.BlockSpec((blk_s, D), lambda i: (i, 0)),
            pl.BlockSpec((E, D), lambda i: (0, 0)),
            pl.BlockSpec((E, D), lambda i: (0, 0)),
            pl.BlockSpec((E, D), lambda i: (0, 0)),
        ],
        out_specs=[
            pl.BlockSpec((blk_s, E), lambda i: (i, 0)),
            pl.BlockSpec((blk_s, 2), lambda i: (i, 0)),
            pl.BlockSpec((blk_s, 2), lambda i: (i, 0)),
        ],
        out_shape=[
            jax.ShapeDtypeStruct((N, E), jnp.float32),
            jax.ShapeDtypeStruct((N, 2), jnp.float32),
            jax.ShapeDtypeStruct((N, 2), jnp.int32),
        ],
    )(x_flat, expert_mus, expert_log_sigmas, sigma)

    mesh = plsc.VectorSubcoreMesh(core_axis_name="c", subcore_axis_name="s",
                                  num_cores=NC, num_subcores=NT // NC)
    counts, ranks = pl.kernel(
        functools.partial(_count_body, ppt, E),
        out_type=[
            jax.ShapeDtypeStruct((NT, L), jnp.int32),
            jax.ShapeDtypeStruct((NT, ppt), jnp.int32),
        ],
        mesh=mesh,
        scratch_types=[
            pltpu.VMEM((ppt // 2, 2), jnp.int32),
            pltpu.VMEM((ppt,), jnp.int32),
            pltpu.VMEM((L,), jnp.int32),
            pltpu.SemaphoreType.DMA,
        ],
        compiler_params=pltpu.CompilerParams(needs_layout_passes=False),
    )(tidx)

    xs, pos3, bex = pl.kernel(
        functools.partial(_dispatch_body, ppt, nch, E, gmax, nbe),
        out_type=[
            jax.ShapeDtypeStruct((gmax, D), jnp.float32),
            jax.ShapeDtypeStruct((NT, nch, 32), jnp.int32),
            jax.ShapeDtypeStruct((nbe,), jnp.int32),
        ],
        mesh=mesh,
        scratch_types=[
            pltpu.VMEM((ppt // 2, 2), jnp.int32),
            pltpu.VMEM((ppt,), jnp.int32),
            pltpu.VMEM((L,), jnp.int32),
            pltpu.VMEM((L,), jnp.int32),
            pltpu.VMEM((NT, L), jnp.int32),
            pltpu.VMEM((nch, 32), jnp.int32),
            pltpu.VMEM((nch, 32), jnp.int32),
            pltpu.VMEM((32, D), jnp.float32),
            pltpu.VMEM((nbe,), jnp.int32),
            pltpu.SemaphoreType.DMA,
            pltpu.SemaphoreType.DMA,
        ],
        compiler_params=pltpu.CompilerParams(needs_layout_passes=False),
    )(tidx, x_flat, counts, ranks)

    blk_h = 512 if H % 512 == 0 else H
    n_h = H // blk_h
    b1r = b1.reshape(E, n_h, 1, blk_h)
    b2r = b2.reshape(E, 1, O)

    out_pairs = pl.pallas_call(
        functools.partial(_mlp_body, n_h=n_h),
        grid_spec=pltpu.PrefetchScalarGridSpec(
            num_scalar_prefetch=1,
            grid=(nb, n_h),
            in_specs=[
                pl.BlockSpec((NBLK, D), lambda i, h, be: (i, 0)),
                pl.BlockSpec((1, D, blk_h), lambda i, h, be: (be[i], 0, h)),
                pl.BlockSpec((1, 1, 1, blk_h), lambda i, h, be: (be[i], h, 0, 0)),
                pl.BlockSpec((1, blk_h, O), lambda i, h, be: (be[i], h, 0)),
                pl.BlockSpec((1, 1, O), lambda i, h, be: (be[i], 0, 0)),
            ],
            out_specs=pl.BlockSpec((NBLK, O), lambda i, h, be: (i, 0)),
            scratch_shapes=[pltpu.VMEM((NBLK, O), jnp.float32)],
        ),
        out_shape=jax.ShapeDtypeStruct((gmax, O), jnp.float32),
        compiler_params=pltpu.CompilerParams(
            dimension_semantics=("arbitrary", "arbitrary")),
    )(bex, xs, W1, b1r, W2, b2r)

    tpt = N // NT
    y = pl.kernel(
        functools.partial(_combine_body, tpt, nch, O),
        out_type=jax.ShapeDtypeStruct((N, O), jnp.float32),
        mesh=mesh,
        scratch_types=[
            pltpu.VMEM((nch, 32), jnp.int32),
            pltpu.VMEM((tpt, 2), jnp.float32),
            pltpu.VMEM((32, O), jnp.float32),
            pltpu.VMEM((tpt // nch, O), jnp.float32),
            pltpu.SemaphoreType.DMA,
        ],
        compiler_params=pltpu.CompilerParams(needs_layout_passes=False),
    )(out_pairs, pos3, wts)

    return (y.reshape(Bn, S, O),
            lp.reshape(Bn, S, E),
            wts.reshape(Bn, S, 2),
            tidx.reshape(Bn, S, 2))


# speedup vs baseline: 1.8897x; 1.2676x over previous
import functools
import math

import jax
import jax.numpy as jnp
from jax import lax
from jax.experimental import pallas as pl
from jax.experimental.pallas import tpu as pltpu
from jax.experimental.pallas import tpu_sc as plsc

L = 16
NC = 2
NT = 32
NBLK = 256
NBLK_LOG = NBLK.bit_length() - 1


def _router_body(x_ref, mus_ref, ls_ref, sig_ref, lp_ref, w_ref, idx_ref):
    x = x_ref[...]
    n_e = mus_ref.shape[0]
    c = 0.5 * math.log(2.0 * math.pi)
    cols = []
    for e in range(n_e):
        t = (x - mus_ref[e][None, :]) / sig_ref[e][None, :]
        term = -0.5 * (t * t) - ls_ref[e][None, :] - c
        cols.append(jnp.sum(term, axis=1, keepdims=True))
    lp = jnp.concatenate(cols, axis=1)
    lp_ref[...] = lp

    rows, e = lp.shape
    iota = lax.broadcasted_iota(jnp.int32, (rows, e), 1)
    v1 = jnp.max(lp, axis=1, keepdims=True)
    i1 = jnp.min(jnp.where(lp == v1, iota, e), axis=1, keepdims=True)
    masked = jnp.where(iota == i1, -jnp.inf, lp)
    v2 = jnp.max(masked, axis=1, keepdims=True)
    i2 = jnp.min(jnp.where(masked == v2, iota, e), axis=1, keepdims=True)
    ew = jnp.exp(v2 - v1)
    w1 = 1.0 / (1.0 + ew)
    w_ref[...] = jnp.concatenate([w1, ew * w1], axis=1)
    idx_ref[...] = jnp.concatenate([i1, i2], axis=1)


def _erf(z):
    a1, a2, a3, a4, a5 = (0.254829592, -0.284496736, 1.421413741,
                          -1.453152027, 1.061405429)
    p = 0.3275911
    s = jnp.sign(z)
    az = jnp.abs(z)
    t = 1.0 / (1.0 + p * az)
    poly = ((((a5 * t + a4) * t + a3) * t + a2) * t + a1) * t
    return s * (1.0 - poly * jnp.exp(-az * az))


def _gelu(v):
    return 0.5 * v * (1.0 + _erf(v * (1.0 / math.sqrt(2.0))))


def _lane_scalar(vec, e):
    io = lax.broadcasted_iota(jnp.int32, (L,), 0)
    return lax.reduce_max(jnp.where(io == e, vec, 0), axes=(0,))


def _count_body(ppt, n_e,
                tidx_hbm, counts_hbm, ranks_hbm,
                e2_vmem, rank_vmem, cnt_vmem, sem1):
    wid = lax.axis_index("c") * (NT // NC) + lax.axis_index("s")
    io = lax.broadcasted_iota(jnp.int32, (L,), 0)
    pltpu.sync_copy(tidx_hbm.at[pl.ds(wid * (ppt // 2), ppt // 2)], e2_vmem)
    cnt = jnp.zeros((L,), jnp.int32)
    nv = ppt // L
    for v in range(nv):
        pj = v * L + io
        ev = plsc.load_gather(e2_vmem, [pj >> 1, pj & 1])
        pre = jnp.zeros((L,), jnp.int32)
        new_cnt = cnt
        for e in range(n_e):
            mi = (ev == e).astype(jnp.int32)
            cs = plsc.cumsum(mi)
            tot = lax.reduce_max(cs, axes=(0,))
            prior = _lane_scalar(cnt, e)
            pre = pre + mi * (cs - 1 + prior)
            new_cnt = new_cnt + tot * (io == e).astype(jnp.int32)
        cnt = new_cnt
        rank_vmem[pl.ds(v * L, L)] = pre
    cnt_vmem[...] = cnt
    pltpu.sync_copy(cnt_vmem, counts_hbm.at[wid])
    pltpu.sync_copy(rank_vmem, ranks_hbm.at[wid])


def _dispatch_body(ppt, nch, n_e, gmax, nbe,
                   tidx_hbm, x_hbm, counts_hbm, ranks_hbm,
                   xs_hbm, pos_hbm, bex_hbm,
                   e2_vmem, rank_vmem, o_vmem, pf_vmem,
                   allc_vmem, pos2_vmem, tok2_vmem, xbuf, bex_vmem,
                   sem1, sem2):
    wid = lax.axis_index("c") * (NT // NC) + lax.axis_index("s")
    base = wid * ppt
    io = lax.broadcasted_iota(jnp.int32, (L,), 0)

    pltpu.sync_copy(tidx_hbm.at[pl.ds(wid * (ppt // 2), ppt // 2)], e2_vmem)
    pltpu.sync_copy(ranks_hbm.at[wid], rank_vmem)
    pltpu.sync_copy(counts_hbm, allc_vmem)

    prefix = jnp.zeros((L,), jnp.int32)
    total = jnp.zeros((L,), jnp.int32)
    for r in range(NT):
        row = allc_vmem[r]
        fl = (jnp.int32(r) < wid).astype(jnp.int32)
        prefix = prefix + row * fl
        total = total + row
    g_vec = ((total + (NBLK - 1)) >> NBLK_LOG) << NBLK_LOG
    o_vec = plsc.cumsum(g_vec) - g_vec
    end_vec = o_vec + g_vec
    o_vmem[...] = o_vec
    pf_vmem[...] = prefix

    nv = ppt // L
    for v in range(nv):
        pj = v * L + io
        ev = plsc.load_gather(e2_vmem, [pj >> 1, pj & 1])
        og = plsc.load_gather(o_vmem, [ev])
        pg = plsc.load_gather(pf_vmem, [ev])
        pos_v = og + pg + rank_vmem[pl.ds(v * L, L)]
        c2, half = divmod(v, 2)
        pos2_vmem[c2, pl.ds(half * L, L)] = pos_v
        tok2_vmem[c2, pl.ds(half * L, L)] = (base + v * L + io) >> 1

    pltpu.sync_copy(pos2_vmem, pos_hbm.at[wid])

    for c in range(nch):
        pltpu.async_copy(x_hbm.at[tok2_vmem.at[c]], xbuf, sem1).wait()
        pltpu.async_copy(xbuf, xs_hbm.at[pos2_vmem.at[c]], sem2).wait()

    @pl.when(wid == 0)
    def _():
        for k in range(nbe // L):
            bstart = (k * L + io) << NBLK_LOG
            acc = jnp.zeros((L,), jnp.int32)
            for e in range(n_e):
                end_e = _lane_scalar(end_vec, e)
                acc = acc + (bstart >= end_e).astype(jnp.int32)
            bex_vmem[pl.ds(k * L, L)] = jnp.minimum(acc, n_e - 1)
        pltpu.sync_copy(bex_vmem, bex_hbm)


def _mlp_body(bex_ref, xs_ref, w1_ref, b1_ref, w2_ref, b2_ref, out_ref,
              acc_ref, *, n_h):
    h = pl.program_id(1)
    ht = _gelu(jnp.dot(xs_ref[...], w1_ref[0],
                       preferred_element_type=jnp.float32) + b1_ref[0, 0])
    part = jnp.dot(ht, w2_ref[0], preferred_element_type=jnp.float32)

    @pl.when(h == 0)
    def _init():
        acc_ref[...] = part

    @pl.when(h != 0)
    def _acc():
        acc_ref[...] += part

    @pl.when(h == n_h - 1)
    def _out():
        out_ref[...] = acc_ref[...] + b2_ref[0]


def _combine_body(tpt, nch, dmodel,
                  op_hbm, pos_hbm, w_hbm, y_hbm,
                  pos2_vmem, w_vmem, rows_v, ybuf, sem1):
    wid = lax.axis_index("c") * (NT // NC) + lax.axis_index("s")
    ct = tpt // nch
    pltpu.sync_copy(pos_hbm.at[wid], pos2_vmem)
    pltpu.sync_copy(w_hbm.at[pl.ds(wid * tpt, tpt)], w_vmem)
    nd = dmodel // L
    io = lax.broadcasted_iota(jnp.int32, (L,), 0)
    for c in range(nch):
        pltpu.async_copy(op_hbm.at[pos2_vmem.at[c]], rows_v, sem1).wait()
        wrow0 = plsc.load_gather(w_vmem, [c * ct + io,
                                          jnp.zeros((L,), jnp.int32)])
        wrow1 = plsc.load_gather(w_vmem, [c * ct + io,
                                          jnp.ones((L,), jnp.int32)])
        for t in range(ct):
            w0 = wrow0[t]
            w1 = wrow1[t]

            def body(d, carry):
                for u in range(4):
                    off = d * (4 * L) + u * L
                    a = rows_v[2 * t, pl.ds(off, L)]
                    b = rows_v[2 * t + 1, pl.ds(off, L)]
                    ybuf[t, pl.ds(off, L)] = w0 * a + w1 * b
                return carry

            lax.fori_loop(0, nd // 4, body, 0)
        pltpu.sync_copy(ybuf, y_hbm.at[pl.ds(wid * tpt + c * ct, ct)])


def kernel(x, expert_mus, expert_log_sigmas, W1, b1, W2, b2):
    Bn, S, D = x.shape
    E, _, H = W1.shape
    O = W2.shape[-1]
    N = Bn * S
    K = 2
    P = N * K
    x_flat = x.reshape(N, D)

    gmax = -(-(P + E * (NBLK - 1)) // NBLK) * NBLK
    nb = gmax // NBLK
    nbe = -(-nb // L) * L
    ppt = P // NT
    nch = ppt // 32
    sigma = jnp.exp(expert_log_sigmas)

    blk_s = 512 if N % 512 == 0 else N
    n_s = N // blk_s
    lp, wts, tidx = pl.pallas_call(
        _router_body,
        grid=(n_s,),
        in_specs=[
            pl.BlockSpec((blk_s, D), lambda i: (i, 0)),
            pl.BlockSpec((E, D), lambda i: (0, 0)),
            pl.BlockSpec((E, D), lambda i: (0, 0)),
            pl.BlockSpec((E, D), lambda i: (0, 0)),
        ],
        out_specs=[
            pl.BlockSpec((blk_s, E), lambda i: (i, 0)),
            pl.BlockSpec((blk_s, 2), lambda i: (i, 0)),
            pl.BlockSpec((blk_s, 2), lambda i: (i, 0)),
        ],
        out_shape=[
            jax.ShapeDtypeStruct((N, E), jnp.float32),
            jax.ShapeDtypeStruct((N, 2), jnp.float32),
            jax.ShapeDtypeStruct((N, 2), jnp.int32),
        ],
    )(x_flat, expert_mus, expert_log_sigmas, sigma)

    mesh = plsc.VectorSubcoreMesh(core_axis_name="c", subcore_axis_name="s",
                                  num_cores=NC, num_subcores=NT // NC)
    counts, ranks = pl.kernel(
        functools.partial(_count_body, ppt, E),
        out_type=[
            jax.ShapeDtypeStruct((NT, L), jnp.int32),
            jax.ShapeDtypeStruct((NT, ppt), jnp.int32),
        ],
        mesh=mesh,
        scratch_types=[
            pltpu.VMEM((ppt // 2, 2), jnp.int32),
            pltpu.VMEM((ppt,), jnp.int32),
            pltpu.VMEM((L,), jnp.int32),
            pltpu.SemaphoreType.DMA,
        ],
        compiler_params=pltpu.CompilerParams(needs_layout_passes=False),
    )(tidx)

    xs, pos3, bex = pl.kernel(
        functools.partial(_dispatch_body, ppt, nch, E, gmax, nbe),
        out_type=[
            jax.ShapeDtypeStruct((gmax, D), jnp.float32),
            jax.ShapeDtypeStruct((NT, nch, 32), jnp.int32),
            jax.ShapeDtypeStruct((nbe,), jnp.int32),
        ],
        mesh=mesh,
        scratch_types=[
            pltpu.VMEM((ppt // 2, 2), jnp.int32),
            pltpu.VMEM((ppt,), jnp.int32),
            pltpu.VMEM((L,), jnp.int32),
            pltpu.VMEM((L,), jnp.int32),
            pltpu.VMEM((NT, L), jnp.int32),
            pltpu.VMEM((nch, 32), jnp.int32),
            pltpu.VMEM((nch, 32), jnp.int32),
            pltpu.VMEM((32, D), jnp.float32),
            pltpu.VMEM((nbe,), jnp.int32),
            pltpu.SemaphoreType.DMA,
            pltpu.SemaphoreType.DMA,
        ],
        compiler_params=pltpu.CompilerParams(needs_layout_passes=False),
    )(tidx, x_flat, counts, ranks)

    blk_h = 512 if H % 512 == 0 else H
    n_h = H // blk_h
    b1r = b1.reshape(E, n_h, 1, blk_h)
    b2r = b2.reshape(E, 1, O)

    out_pairs = pl.pallas_call(
        functools.partial(_mlp_body, n_h=n_h),
        grid_spec=pltpu.PrefetchScalarGridSpec(
            num_scalar_prefetch=1,
            grid=(nb, n_h),
            in_specs=[
                pl.BlockSpec((NBLK, D), lambda i, h, be: (i, 0)),
                pl.BlockSpec((1, D, blk_h), lambda i, h, be: (be[i], 0, h)),
                pl.BlockSpec((1, 1, 1, blk_h), lambda i, h, be: (be[i], h, 0, 0)),
                pl.BlockSpec((1, blk_h, O), lambda i, h, be: (be[i], h, 0)),
                pl.BlockSpec((1, 1, O), lambda i, h, be: (be[i], 0, 0)),
            ],
            out_specs=pl.BlockSpec((NBLK, O), lambda i, h, be: (i, 0)),
            scratch_shapes=[pltpu.VMEM((NBLK, O), jnp.float32)],
        ),
        out_shape=jax.ShapeDtypeStruct((gmax, O), jnp.float32),
        compiler_params=pltpu.CompilerParams(
            dimension_semantics=("arbitrary", "arbitrary")),
    )(bex, xs, W1, b1r, W2, b2r)

    tpt = N // NT
    y = pl.kernel(
        functools.partial(_combine_body, tpt, nch, O),
        out_type=jax.ShapeDtypeStruct((N, O), jnp.float32),
        mesh=mesh,
        scratch_types=[
            pltpu.VMEM((nch, 32), jnp.int32),
            pltpu.VMEM((tpt, 2), jnp.float32),
            pltpu.VMEM((32, O), jnp.float32),
            pltpu.VMEM((tpt // nch, O), jnp.float32),
            pltpu.SemaphoreType.DMA,
        ],
        compiler_params=pltpu.CompilerParams(needs_layout_passes=False),
    )(out_pairs, pos3, wts)

    return (y.reshape(Bn, S, O),
            lp.reshape(Bn, S, E),
            wts.reshape(Bn, S, 2),
            tidx.reshape(Bn, S, 2))


# speedup vs baseline: 2.1592x; 1.1427x over previous
import functools
import math

import jax
import jax.numpy as jnp
from jax import lax
from jax.experimental import pallas as pl
from jax.experimental.pallas import tpu as pltpu
from jax.experimental.pallas import tpu_sc as plsc

L = 16
NC = 2
NT = 32
NBLK = 256
NBLK_LOG = NBLK.bit_length() - 1


def _router_body(x_ref, mus_ref, ls_ref, sig_ref, lp_ref, w_ref, idx_ref):
    x = x_ref[...]
    n_e = mus_ref.shape[0]
    c = 0.5 * math.log(2.0 * math.pi)
    cols = []
    for e in range(n_e):
        t = (x - mus_ref[e][None, :]) / sig_ref[e][None, :]
        term = -0.5 * (t * t) - ls_ref[e][None, :] - c
        cols.append(jnp.sum(term, axis=1, keepdims=True))
    lp = jnp.concatenate(cols, axis=1)
    lp_ref[...] = lp

    rows, e = lp.shape
    iota = lax.broadcasted_iota(jnp.int32, (rows, e), 1)
    v1 = jnp.max(lp, axis=1, keepdims=True)
    i1 = jnp.min(jnp.where(lp == v1, iota, e), axis=1, keepdims=True)
    masked = jnp.where(iota == i1, -jnp.inf, lp)
    v2 = jnp.max(masked, axis=1, keepdims=True)
    i2 = jnp.min(jnp.where(masked == v2, iota, e), axis=1, keepdims=True)
    ew = jnp.exp(v2 - v1)
    w1 = 1.0 / (1.0 + ew)
    w_ref[...] = jnp.concatenate([w1, ew * w1], axis=1)
    idx_ref[...] = jnp.concatenate([i1, i2], axis=1)


def _erf(z):
    a1, a2, a3, a4, a5 = (0.254829592, -0.284496736, 1.421413741,
                          -1.453152027, 1.061405429)
    p = 0.3275911
    s = jnp.sign(z)
    az = jnp.abs(z)
    t = 1.0 / (1.0 + p * az)
    poly = ((((a5 * t + a4) * t + a3) * t + a2) * t + a1) * t
    return s * (1.0 - poly * jnp.exp(-az * az))


def _gelu(v):
    return 0.5 * v * (1.0 + _erf(v * (1.0 / math.sqrt(2.0))))


def _lane_scalar(vec, e):
    io = lax.broadcasted_iota(jnp.int32, (L,), 0)
    return lax.reduce_max(jnp.where(io == e, vec, 0), axes=(0,))


def _count_body(ppt, n_e,
                tidx_hbm, counts_hbm, ranks_hbm,
                e2_vmem, rank_vmem, cnt_vmem, sem1):
    wid = lax.axis_index("c") * (NT // NC) + lax.axis_index("s")
    io = lax.broadcasted_iota(jnp.int32, (L,), 0)
    pltpu.sync_copy(tidx_hbm.at[pl.ds(wid * (ppt // 2), ppt // 2)], e2_vmem)
    cnt = jnp.zeros((L,), jnp.int32)
    nv = ppt // L
    for v in range(nv):
        pj = v * L + io
        ev = plsc.load_gather(e2_vmem, [pj >> 1, pj & 1])
        pre = jnp.zeros((L,), jnp.int32)
        new_cnt = cnt
        for e in range(n_e):
            mi = (ev == e).astype(jnp.int32)
            cs = plsc.cumsum(mi)
            tot = lax.reduce_max(cs, axes=(0,))
            prior = _lane_scalar(cnt, e)
            pre = pre + mi * (cs - 1 + prior)
            new_cnt = new_cnt + tot * (io == e).astype(jnp.int32)
        cnt = new_cnt
        rank_vmem[pl.ds(v * L, L)] = pre
    cnt_vmem[...] = cnt
    pltpu.sync_copy(cnt_vmem, counts_hbm.at[wid])
    pltpu.sync_copy(rank_vmem, ranks_hbm.at[wid])


def _dispatch_body(ppt, nch, n_e, gmax, nbe,
                   tidx_hbm, x_hbm, counts_hbm, ranks_hbm,
                   xs_hbm, pos_hbm, bex_hbm,
                   e2_vmem, rank_vmem, o_vmem, pf_vmem,
                   allc_vmem, pos2_vmem, tok2_vmem, xbuf, bex_vmem,
                   sem1, sem2):
    wid = lax.axis_index("c") * (NT // NC) + lax.axis_index("s")
    base = wid * ppt
    io = lax.broadcasted_iota(jnp.int32, (L,), 0)

    pltpu.sync_copy(tidx_hbm.at[pl.ds(wid * (ppt // 2), ppt // 2)], e2_vmem)
    pltpu.sync_copy(ranks_hbm.at[wid], rank_vmem)
    pltpu.sync_copy(counts_hbm, allc_vmem)

    prefix = jnp.zeros((L,), jnp.int32)
    total = jnp.zeros((L,), jnp.int32)
    for r in range(NT):
        row = allc_vmem[r]
        fl = (jnp.int32(r) < wid).astype(jnp.int32)
        prefix = prefix + row * fl
        total = total + row
    g_vec = ((total + (NBLK - 1)) >> NBLK_LOG) << NBLK_LOG
    o_vec = plsc.cumsum(g_vec) - g_vec
    end_vec = o_vec + g_vec
    o_vmem[...] = o_vec
    pf_vmem[...] = prefix

    nv = ppt // L
    for v in range(nv):
        pj = v * L + io
        ev = plsc.load_gather(e2_vmem, [pj >> 1, pj & 1])
        og = plsc.load_gather(o_vmem, [ev])
        pg = plsc.load_gather(pf_vmem, [ev])
        pos_v = og + pg + rank_vmem[pl.ds(v * L, L)]
        c2, half = divmod(v, 2)
        pos2_vmem[c2, pl.ds(half * L, L)] = pos_v
        tok2_vmem[c2, pl.ds(half * L, L)] = (base + v * L + io) >> 1

    pltpu.sync_copy(pos2_vmem, pos_hbm.at[wid])

    for c in range(nch):
        pltpu.async_copy(x_hbm.at[tok2_vmem.at[c]], xbuf, sem1).wait()
        pltpu.async_copy(xbuf, xs_hbm.at[pos2_vmem.at[c]], sem2).wait()

    @pl.when(wid == 0)
    def _():
        for k in range(nbe // L):
            bstart = (k * L + io) << NBLK_LOG
            acc = jnp.zeros((L,), jnp.int32)
            for e in range(n_e):
                end_e = _lane_scalar(end_vec, e)
                acc = acc + (bstart >= end_e).astype(jnp.int32)
            bex_vmem[pl.ds(k * L, L)] = jnp.minimum(acc, n_e - 1)
        pltpu.sync_copy(bex_vmem, bex_hbm)


def _mlp_body(bex_ref, xs_ref, w1_ref, b1_ref, w2_ref, b2_ref, out_ref,
              acc_ref, *, n_h):
    h = pl.program_id(1)
    ht = _gelu(jnp.dot(xs_ref[...], w1_ref[0],
                       preferred_element_type=jnp.float32) + b1_ref[0, 0])
    part = jnp.dot(ht, w2_ref[0], preferred_element_type=jnp.float32)

    @pl.when(h == 0)
    def _init():
        acc_ref[...] = part

    @pl.when(h != 0)
    def _acc():
        acc_ref[...] += part

    @pl.when(h == n_h - 1)
    def _out():
        out_ref[...] = acc_ref[...] + b2_ref[0]


def _combine_body(tpt, nch, dmodel,
                  op_hbm, pos_hbm, w_hbm, y_hbm,
                  pos2_vmem, w_vmem, rows_v, ybuf, sem1):
    wid = lax.axis_index("c") * (NT // NC) + lax.axis_index("s")
    ct = tpt // nch
    pltpu.sync_copy(pos_hbm.at[wid], pos2_vmem)
    pltpu.sync_copy(w_hbm.at[pl.ds(wid * tpt, tpt)], w_vmem)
    nd = dmodel // L
    io = lax.broadcasted_iota(jnp.int32, (L,), 0)
    for c in range(nch):
        pltpu.async_copy(op_hbm.at[pos2_vmem.at[c]], rows_v, sem1).wait()
        wrow0 = plsc.load_gather(w_vmem, [c * ct + io,
                                          jnp.zeros((L,), jnp.int32)])
        wrow1 = plsc.load_gather(w_vmem, [c * ct + io,
                                          jnp.ones((L,), jnp.int32)])
        for t in range(ct):
            w0 = wrow0[t]
            w1 = wrow1[t]

            def body(d, carry):
                for u in range(4):
                    off = d * (4 * L) + u * L
                    a = rows_v[2 * t, pl.ds(off, L)]
                    b = rows_v[2 * t + 1, pl.ds(off, L)]
                    ybuf[t, pl.ds(off, L)] = w0 * a + w1 * b
                return carry

            lax.fori_loop(0, nd // 4, body, 0)
        pltpu.sync_copy(ybuf, y_hbm.at[pl.ds(wid * tpt + c * ct, ct)])


def kernel(x, expert_mus, expert_log_sigmas, W1, b1, W2, b2):
    Bn, S, D = x.shape
    E, _, H = W1.shape
    O = W2.shape[-1]
    N = Bn * S
    K = 2
    P = N * K
    x_flat = x.reshape(N, D)

    gmax = -(-(P + E * (NBLK - 1)) // NBLK) * NBLK
    nb = gmax // NBLK
    nbe = -(-nb // L) * L
    ppt = P // NT
    nch = ppt // 32
    sigma = jnp.exp(expert_log_sigmas)

    blk_s = 512 if N % 512 == 0 else N
    n_s = N // blk_s
    lp, wts, tidx = pl.pallas_call(
        _router_body,
        grid=(n_s,),
        in_specs=[
            pl.BlockSpec((blk_s, D), lambda i: (i, 0)),
            pl.BlockSpec((E, D), lambda i: (0, 0)),
            pl.BlockSpec((E, D), lambda i: (0, 0)),
            pl.BlockSpec((E, D), lambda i: (0, 0)),
        ],
        out_specs=[
            pl.BlockSpec((blk_s, E), lambda i: (i, 0)),
            pl.BlockSpec((blk_s, 2), lambda i: (i, 0)),
            pl.BlockSpec((blk_s, 2), lambda i: (i, 0)),
        ],
        out_shape=[
            jax.ShapeDtypeStruct((N, E), jnp.float32),
            jax.ShapeDtypeStruct((N, 2), jnp.float32),
            jax.ShapeDtypeStruct((N, 2), jnp.int32),
        ],
    )(x_flat, expert_mus, expert_log_sigmas, sigma)

    mesh = plsc.VectorSubcoreMesh(core_axis_name="c", subcore_axis_name="s",
                                  num_cores=NC, num_subcores=NT // NC)
    counts, ranks = pl.kernel(
        functools.partial(_count_body, ppt, E),
        out_type=[
            jax.ShapeDtypeStruct((NT, L), jnp.int32),
            jax.ShapeDtypeStruct((NT, ppt), jnp.int32),
        ],
        mesh=mesh,
        scratch_types=[
            pltpu.VMEM((ppt // 2, 2), jnp.int32),
            pltpu.VMEM((ppt,), jnp.int32),
            pltpu.VMEM((L,), jnp.int32),
            pltpu.SemaphoreType.DMA,
        ],
        compiler_params=pltpu.CompilerParams(needs_layout_passes=False),
    )(tidx)

    xs, pos3, bex = pl.kernel(
        functools.partial(_dispatch_body, ppt, nch, E, gmax, nbe),
        out_type=[
            jax.ShapeDtypeStruct((gmax, D), jnp.float32),
            jax.ShapeDtypeStruct((NT, nch, 32), jnp.int32),
            jax.ShapeDtypeStruct((nbe,), jnp.int32),
        ],
        mesh=mesh,
        scratch_types=[
            pltpu.VMEM((ppt // 2, 2), jnp.int32),
            pltpu.VMEM((ppt,), jnp.int32),
            pltpu.VMEM((L,), jnp.int32),
            pltpu.VMEM((L,), jnp.int32),
            pltpu.VMEM((NT, L), jnp.int32),
            pltpu.VMEM((nch, 32), jnp.int32),
            pltpu.VMEM((nch, 32), jnp.int32),
            pltpu.VMEM((32, D), jnp.float32),
            pltpu.VMEM((nbe,), jnp.int32),
            pltpu.SemaphoreType.DMA,
            pltpu.SemaphoreType.DMA,
        ],
        compiler_params=pltpu.CompilerParams(needs_layout_passes=False),
    )(tidx, x_flat, counts, ranks)

    blk_h = 1024 if H % 1024 == 0 else H
    n_h = H // blk_h
    b1r = b1.reshape(E, n_h, 1, blk_h)
    b2r = b2.reshape(E, 1, O)

    out_pairs = pl.pallas_call(
        functools.partial(_mlp_body, n_h=n_h),
        grid_spec=pltpu.PrefetchScalarGridSpec(
            num_scalar_prefetch=1,
            grid=(nb, n_h),
            in_specs=[
                pl.BlockSpec((NBLK, D), lambda i, h, be: (i, 0)),
                pl.BlockSpec((1, D, blk_h), lambda i, h, be: (be[i], 0, h)),
                pl.BlockSpec((1, 1, 1, blk_h), lambda i, h, be: (be[i], h, 0, 0)),
                pl.BlockSpec((1, blk_h, O), lambda i, h, be: (be[i], h, 0)),
                pl.BlockSpec((1, 1, O), lambda i, h, be: (be[i], 0, 0)),
            ],
            out_specs=pl.BlockSpec((NBLK, O), lambda i, h, be: (i, 0)),
            scratch_shapes=[pltpu.VMEM((NBLK, O), jnp.float32)],
        ),
        out_shape=jax.ShapeDtypeStruct((gmax, O), jnp.float32),
        compiler_params=pltpu.CompilerParams(
            dimension_semantics=("arbitrary", "arbitrary")),
    )(bex, xs, W1, b1r, W2, b2r)

    tpt = N // NT
    y = pl.kernel(
        functools.partial(_combine_body, tpt, nch, O),
        out_type=jax.ShapeDtypeStruct((N, O), jnp.float32),
        mesh=mesh,
        scratch_types=[
            pltpu.VMEM((nch, 32), jnp.int32),
            pltpu.VMEM((tpt, 2), jnp.float32),
            pltpu.VMEM((32, O), jnp.float32),
            pltpu.VMEM((tpt // nch, O), jnp.float32),
            pltpu.SemaphoreType.DMA,
        ],
        compiler_params=pltpu.CompilerParams(needs_layout_passes=False),
    )(out_pairs, pos3, wts)

    return (y.reshape(Bn, S, O),
            lp.reshape(Bn, S, E),
            wts.reshape(Bn, S, 2),
            tidx.reshape(Bn, S, 2))
